# Initial kernel scaffold; baseline (speedup 1.0000x reference)
import jax
import jax.numpy as jnp
from jax import lax
from jax.experimental import pallas as pl
from jax.experimental.pallas import tpu as pltpu
from jax.experimental.pallas import tpu_sc as plsc

NUM_CORES = 2
NUM_SUBCORES = 16
NUM_WORKERS = NUM_CORES * NUM_SUBCORES

B = 16384 * 50
D = 64
BPW = B // NUM_WORKERS
CHUNK = 512
NCHUNKS = BPW // CHUNK


def _gather_body(table_hbm, idx_hbm, out_hbm, idx_v, rows, sem):
    wid = lax.axis_index("s") * NUM_CORES + lax.axis_index("c")
    base = wid * BPW
    pltpu.sync_copy(idx_hbm.at[pl.ds(base, BPW)], idx_v)

    @pl.loop(0, NCHUNKS)
    def _(j):
        off = j * CHUNK
        pltpu.async_copy(
            table_hbm.at[idx_v.at[pl.ds(off, CHUNK)]], rows, sem
        ).wait()
        pltpu.sync_copy(rows, out_hbm.at[pl.ds(base + off, CHUNK)])


@jax.jit
def _gather(table, idx_flat):
    mesh = plsc.VectorSubcoreMesh(
        core_axis_name="c",
        subcore_axis_name="s",
        num_cores=NUM_CORES,
        num_subcores=NUM_SUBCORES,
    )
    fn = pl.kernel(
        _gather_body,
        out_type=jax.ShapeDtypeStruct((B, D), jnp.float32),
        mesh=mesh,
        scratch_types=[
            pltpu.VMEM((BPW,), jnp.int32),
            pltpu.VMEM((CHUNK, D), jnp.float32),
            pltpu.SemaphoreType.DMA,
        ],
    )
    return fn(table, idx_flat)


def kernel(x, table):
    idx_flat = x.reshape(-1).astype(jnp.int32)
    out = _gather(table, idx_flat)
    return out.reshape(x.shape + (D,))


# speedup vs baseline: 1.8349x; 1.8349x over previous
import jax
import jax.numpy as jnp
from jax import lax
from jax.experimental import pallas as pl
from jax.experimental.pallas import tpu as pltpu
from jax.experimental.pallas import tpu_sc as plsc

NUM_CORES = 2
NUM_SUBCORES = 16
NUM_WORKERS = NUM_CORES * NUM_SUBCORES

B = 16384 * 50
D = 64
BPW = B // NUM_WORKERS
CHUNK = 512
NCHUNKS = BPW // CHUNK


def _gather_body(table_hbm, idx_hbm, out_hbm, idx_v, rows, sem):
    wid = lax.axis_index("s") * NUM_CORES + lax.axis_index("c")
    base = wid * BPW
    pltpu.sync_copy(idx_hbm.at[pl.ds(base, BPW)], idx_v)

    @pl.loop(0, NCHUNKS)
    def _(j):
        off = j * CHUNK
        pltpu.async_copy(
            table_hbm.at[idx_v.at[pl.ds(off, CHUNK)]], rows, sem
        ).wait()
        pltpu.sync_copy(rows, out_hbm.at[pl.ds(base + off, CHUNK)])


@jax.jit
def _gather(table, idx_flat):
    mesh = plsc.VectorSubcoreMesh(
        core_axis_name="c",
        subcore_axis_name="s",
        num_cores=NUM_CORES,
        num_subcores=NUM_SUBCORES,
    )
    fn = pl.kernel(
        _gather_body,
        out_type=jax.ShapeDtypeStruct((B, D), jnp.float32),
        mesh=mesh,
        compiler_params=pltpu.CompilerParams(use_tc_tiling_on_sc=False),
        scratch_types=[
            pltpu.VMEM((BPW,), jnp.int32),
            pltpu.VMEM((CHUNK, D), jnp.float32),
            pltpu.SemaphoreType.DMA,
        ],
    )
    return fn(table, idx_flat)


def kernel(x, table):
    idx_flat = x.reshape(-1).astype(jnp.int32)
    out = _gather(table, idx_flat)
    return out.reshape(x.shape + (D,))
